# baseline (speedup 1.0000x reference)
import functools

import jax
import jax.numpy as jnp
from jax import lax
from jax.experimental import pallas as pl
from jax.experimental.pallas import tpu as pltpu
from jax.experimental.pallas import tpu_sc as plsc

_BATCH = 4096
_HIST = 50
_D = 64
_NL = 1000000
_FULL = 999936
_WLAN = 384
_NWIN = _FULL // _WLAN
_WWORDS = _WLAN * _D
_NPAIR = 41


@functools.lru_cache(maxsize=None)
def _build_detile():
    info = plsc.get_sparse_core_info()
    nc, ns = info.num_cores, info.num_subcores
    nw = nc * ns
    mesh = plsc.VectorSubcoreMesh(core_axis_name="c", subcore_axis_name="s")

    @functools.partial(
        pl.kernel,
        out_type=(
            jax.ShapeDtypeStruct((_NL * _D,), jnp.float32),
            jax.ShapeDtypeStruct((_NL * _D,), jnp.float32),
        ),
        mesh=mesh,
        compiler_params=pltpu.CompilerParams(needs_layout_passes=False),
        scratch_types=[
            [pltpu.VMEM((24 * 8, 128), jnp.float32) for _ in range(2)],
            [pltpu.VMEM((_WWORDS,), jnp.float32) for _ in range(2)],
            [pltpu.SemaphoreType.DMA for _ in range(2)],
            [pltpu.SemaphoreType.DMA for _ in range(2)],
        ],
    )
    def detile(itab_t, utab_t, itail, utail, iscr, uscr,
               stgs, orows, ssems, osems):
        cid = lax.axis_index("c")
        sid = lax.axis_index("s")
        wid = sid * nc + cid

        j = lax.iota(jnp.int32, 16)
        fpats = [((2 * k + (j >> 3)) * 3072 + (j & 7) * 128).astype(jnp.int32)
                 for k in range(4)]

        def lane0_of(i):
            return lax.rem(wid + i * nw, _NWIN) * _WLAN

        def stage(tab_t, b, i, fire):
            l0 = lane0_of(i)
            for g in range(8):
                for c in range(3):
                    src = tab_t.at[pl.ds(g * 8, 8), pl.ds(l0 + c * 128, 128)]
                    dst = stgs[b].at[pl.ds((g * 3 + c) * 8, 8)]
                    if fire:
                        pltpu.async_copy(src, dst, ssems[b])
                    else:
                        pltpu.make_async_copy(src, dst, ssems[b]).wait()

        def compute(b):
            @pl.loop(0, _WLAN, unroll=4)
            def _row(l):
                lt = (l >> 7) * 1024 + (l & 127)
                for k in range(4):
                    idx = fpats[k] + lt
                    vals = plsc.load_gather(stgs[b], [idx >> 7, idx & 127])
                    orows[b][pl.ds(l * _D + k * 16, 16)] = vals

        def fire_out(scr, b, i):
            pltpu.async_copy(
                orows[b], scr.at[pl.ds(lane0_of(i) * _D, _WWORDS)], osems[b])

        def wait_out(scr, b, i):
            pltpu.make_async_copy(
                orows[b], scr.at[pl.ds(lane0_of(i) * _D, _WWORDS)],
                osems[b]).wait()

        def do_table(tab_t, scr):
            stage(tab_t, 0, 0, True)

            @pl.loop(0, _NPAIR)
            def _pair(i):
                i0 = 2 * i
                i1 = 2 * i + 1
                stage(tab_t, 0, i0, False)
                stage(tab_t, 1, i1, True)

                @pl.when(i > 0)
                def _():
                    wait_out(scr, 0, i0 - 2)
                compute(0)
                fire_out(scr, 0, i0)

                stage(tab_t, 1, i1, False)

                @pl.when(i < _NPAIR - 1)
                def _():
                    stage(tab_t, 0, i0 + 2, True)

                @pl.when(i > 0)
                def _():
                    wait_out(scr, 1, i1 - 2)
                compute(1)
                fire_out(scr, 1, i1)

            wait_out(scr, 0, 2 * _NPAIR - 2)
            wait_out(scr, 1, 2 * _NPAIR - 1)

        do_table(itab_t, iscr)
        do_table(utab_t, uscr)

        @pl.when(wid == 0)
        def _():
            pltpu.sync_copy(itail, orows[0].at[pl.ds(0, 64 * _D)])
            pltpu.sync_copy(orows[0].at[pl.ds(0, 64 * _D)],
                            iscr.at[pl.ds(_FULL * _D, 64 * _D)])

        @pl.when(wid == 1)
        def _():
            pltpu.sync_copy(utail, orows[0].at[pl.ds(0, 64 * _D)])
            pltpu.sync_copy(orows[0].at[pl.ds(0, 64 * _D)],
                            uscr.at[pl.ds(_FULL * _D, 64 * _D)])

    return detile


@functools.lru_cache(maxsize=None)
def _build_gather():
    info = plsc.get_sparse_core_info()
    nc, ns = info.num_cores, info.num_subcores
    nw = nc * ns
    ub = _BATCH // nw
    ib = _BATCH * _HIST // nw
    chunk = 128
    nchunk = ib // chunk
    nbuf = 5
    ngrp = nchunk // nbuf

    mesh = plsc.VectorSubcoreMesh(core_axis_name="c", subcore_axis_name="s")

    @functools.partial(
        pl.kernel,
        out_type=(
            jax.ShapeDtypeStruct((_BATCH, _D), jnp.float32),
            jax.ShapeDtypeStruct((_BATCH * _HIST, _D), jnp.float32),
        ),
        mesh=mesh,
        compiler_params=pltpu.CompilerParams(use_tc_tiling_on_sc=False),
        scratch_types=[
            pltpu.VMEM((ub,), jnp.int32),
            pltpu.VMEM((ub, _D), jnp.float32),
            pltpu.VMEM((ib,), jnp.int32),
            [pltpu.VMEM((chunk, _D), jnp.float32) for _ in range(nbuf)],
            pltpu.SemaphoreType.DMA,
            [pltpu.SemaphoreType.DMA for _ in range(nbuf)],
            [pltpu.SemaphoreType.DMA for _ in range(nbuf)],
        ],
    )
    def emb(uid, iid, utab, itab, uout, iout,
            uidx, urows, iidx, bufs, usem, gsems, wsems):
        wid = lax.axis_index("s") * nc + lax.axis_index("c")
        ubase = wid * ub
        ibase = wid * ib

        pltpu.sync_copy(uid.at[pl.ds(ubase, ub)], uidx)
        pltpu.async_copy(utab.at[uidx], urows, usem)
        pltpu.sync_copy(iid.at[pl.ds(ibase, ib)], iidx)
        pltpu.make_async_copy(utab.at[uidx], urows, usem).wait()
        pltpu.async_copy(urows, uout.at[pl.ds(ubase, ub)], usem)

        def gather(c, b):
            pltpu.async_copy(
                itab.at[iidx.at[pl.ds(c * chunk, chunk)]], bufs[b], gsems[b])

        def wait_gather(c, b):
            pltpu.make_async_copy(
                itab.at[iidx.at[pl.ds(c * chunk, chunk)]], bufs[b],
                gsems[b]).wait()

        def put(c, b):
            pltpu.async_copy(
                bufs[b], iout.at[pl.ds(ibase + c * chunk, chunk)], wsems[b])

        def wait_put(c, b):
            pltpu.make_async_copy(
                bufs[b], iout.at[pl.ds(ibase + c * chunk, chunk)],
                wsems[b]).wait()

        for b in range(nbuf):
            gather(b, b)

        @pl.loop(0, ngrp - 1)
        def _grp(g):
            c0 = g * nbuf
            for b in range(nbuf):
                wait_gather(c0 + b, b)
                put(c0 + b, b)
                wait_put(c0 + b, b)
                gather(c0 + nbuf + b, b)

        c0 = (ngrp - 1) * nbuf
        for b in range(nbuf):
            wait_gather(c0 + b, b)
            put(c0 + b, b)
            wait_put(c0 + b, b)

        pltpu.make_async_copy(urows, uout.at[pl.ds(ubase, ub)], usem).wait()

    return emb


def kernel(user_id, items_ids, user_table, item_table):
    detile = _build_detile()
    emb = _build_gather()
    uid = user_id.astype(jnp.int32)
    iid = items_ids.reshape(-1).astype(jnp.int32)
    itail = item_table[_FULL:].reshape(-1)
    utail = user_table[_FULL:].reshape(-1)
    iscr, uscr = detile(item_table.T, user_table.T, itail, utail)
    user_eb, item_flat = emb(uid, iid,
                             uscr.reshape(_NL, _D), iscr.reshape(_NL, _D))
    return user_eb, item_flat.reshape(_BATCH, _HIST, _D)


# speedup vs baseline: 1.2307x; 1.2307x over previous
import functools

import jax
import jax.numpy as jnp
from jax import lax
from jax.experimental import pallas as pl
from jax.experimental.pallas import tpu as pltpu
from jax.experimental.pallas import tpu_sc as plsc

_BATCH = 4096
_HIST = 50
_D = 64
_NL = 1000000
_BL = 256
_S = 500224
_NBLK = _S // _BL
_SROWS = 2 * _S


@functools.lru_cache(maxsize=None)
def _build_transpose():
    def body(lo_ref, hi_ref, out_ref):
        lo = jnp.transpose(lo_ref[...])
        hi = jnp.transpose(hi_ref[...])
        out_ref[...] = jnp.concatenate([lo, hi], axis=1)

    return pl.pallas_call(
        body,
        grid=(_NBLK,),
        in_specs=[
            pl.BlockSpec((_D, _BL), lambda j: (0, j)),
            pl.BlockSpec((_D, _BL),
                         lambda j: (0, jnp.minimum(j + _NBLK,
                                                   (_NL - 1) // _BL))),
        ],
        out_specs=pl.BlockSpec((_BL, 128), lambda j: (j, 0)),
        out_shape=jax.ShapeDtypeStruct((_S, 128), jnp.float32),
        compiler_params=pltpu.CompilerParams(
            dimension_semantics=("arbitrary",)),
    )


@functools.lru_cache(maxsize=None)
def _build_gather():
    info = plsc.get_sparse_core_info()
    nc, ns = info.num_cores, info.num_subcores
    nw = nc * ns
    ub = _BATCH // nw
    ib = _BATCH * _HIST // nw
    chunk = 128
    nchunk = ib // chunk
    nbuf = 5
    ngrp = nchunk // nbuf

    mesh = plsc.VectorSubcoreMesh(core_axis_name="c", subcore_axis_name="s")

    @functools.partial(
        pl.kernel,
        out_type=(
            jax.ShapeDtypeStruct((_BATCH, _D), jnp.float32),
            jax.ShapeDtypeStruct((_BATCH * _HIST, _D), jnp.float32),
        ),
        mesh=mesh,
        compiler_params=pltpu.CompilerParams(use_tc_tiling_on_sc=False),
        scratch_types=[
            pltpu.VMEM((ub,), jnp.int32),
            pltpu.VMEM((ub, _D), jnp.float32),
            pltpu.VMEM((ib,), jnp.int32),
            [pltpu.VMEM((chunk, _D), jnp.float32) for _ in range(nbuf)],
            pltpu.SemaphoreType.DMA,
            [pltpu.SemaphoreType.DMA for _ in range(nbuf)],
            [pltpu.SemaphoreType.DMA for _ in range(nbuf)],
        ],
    )
    def emb(uid, iid, utab, itab, uout, iout,
            uidx, urows, iidx, bufs, usem, gsems, wsems):
        wid = lax.axis_index("s") * nc + lax.axis_index("c")
        ubase = wid * ub
        ibase = wid * ib

        pltpu.sync_copy(uid.at[pl.ds(ubase, ub)], uidx)

        @pl.loop(0, ub // 16)
        def _tu(t):
            iv = uidx[pl.ds(t * 16, 16)]
            uidx[pl.ds(t * 16, 16)] = jnp.where(
                iv < _S, 2 * iv, 2 * (iv - _S) + 1)

        pltpu.async_copy(utab.at[uidx], urows, usem)
        pltpu.sync_copy(iid.at[pl.ds(ibase, ib)], iidx)

        @pl.loop(0, ib // 16)
        def _ti(t):
            iv = iidx[pl.ds(t * 16, 16)]
            iidx[pl.ds(t * 16, 16)] = jnp.where(
                iv < _S, 2 * iv, 2 * (iv - _S) + 1)

        pltpu.make_async_copy(utab.at[uidx], urows, usem).wait()
        pltpu.async_copy(urows, uout.at[pl.ds(ubase, ub)], usem)

        def gather(c, b):
            pltpu.async_copy(
                itab.at[iidx.at[pl.ds(c * chunk, chunk)]], bufs[b], gsems[b])

        def wait_gather(c, b):
            pltpu.make_async_copy(
                itab.at[iidx.at[pl.ds(c * chunk, chunk)]], bufs[b],
                gsems[b]).wait()

        def put(c, b):
            pltpu.async_copy(
                bufs[b], iout.at[pl.ds(ibase + c * chunk, chunk)], wsems[b])

        def wait_put(c, b):
            pltpu.make_async_copy(
                bufs[b], iout.at[pl.ds(ibase + c * chunk, chunk)],
                wsems[b]).wait()

        for b in range(nbuf):
            gather(b, b)

        @pl.loop(0, ngrp - 1)
        def _grp(g):
            c0 = g * nbuf
            for b in range(nbuf):
                wait_gather(c0 + b, b)
                put(c0 + b, b)
                wait_put(c0 + b, b)
                gather(c0 + nbuf + b, b)

        c0 = (ngrp - 1) * nbuf
        for b in range(nbuf):
            wait_gather(c0 + b, b)
            put(c0 + b, b)
            wait_put(c0 + b, b)

        pltpu.make_async_copy(urows, uout.at[pl.ds(ubase, ub)], usem).wait()

    return emb


def kernel(user_id, items_ids, user_table, item_table):
    tposer = _build_transpose()
    emb = _build_gather()
    uid = user_id.astype(jnp.int32)
    iid = items_ids.reshape(-1).astype(jnp.int32)
    iscr = tposer(item_table.T, item_table.T)
    uscr = tposer(user_table.T, user_table.T)
    user_eb, item_flat = emb(uid, iid,
                             uscr.reshape(_SROWS, _D), iscr.reshape(_SROWS, _D))
    return user_eb, item_flat.reshape(_BATCH, _HIST, _D)
